# baseline (speedup 1.0000x reference)
import functools

import jax
import jax.numpy as jnp
from jax import lax
from jax.experimental import pallas as pl
from jax.experimental.pallas import tpu as pltpu
from jax.experimental.pallas import tpu_sc as plsc

_B = 8
_N = 16384
_NPT = 1024
_TPB = 4
_CHUNK = _N // _TPB
_L = 16
_NVEC = _CHUNK // _L


def _fps_body(xyz_hbm, out_hbm, xyz_v, dists_v, rec_v, comb_v, out_v, shared):
    cid = lax.axis_index("c")
    sid = lax.axis_index("s")
    lb = sid // _TPB
    t = sid % _TPB
    b = cid * 4 + lb
    base = t * _CHUNK

    lane_iota = lax.iota(jnp.int32, _L)
    lane0 = lane_iota == 0
    f_iota = lane_iota.astype(jnp.float32)
    zeros_i = jnp.zeros((_L,), jnp.int32)
    ones_i = jnp.full((_L,), 1, jnp.int32)
    twos_i = jnp.full((_L,), 2, jnp.int32)

    tz = lax.shift_right_logical(sid, 5)

    pltpu.sync_copy(xyz_hbm.at[b], xyz_v)

    def _init(i, c):
        dists_v[pl.ds(i * _L, _L)] = jnp.full((_L,), 1e10, jnp.float32)
        return c
    lax.fori_loop(0, _NVEC, _init, 0)

    acc0 = jnp.zeros((_L,), jnp.int32)

    def _lane0_splat(row):
        v = xyz_v[row, pl.ds(tz, _L)]
        return jnp.broadcast_to(jnp.sum(jnp.where(lane0, v, 0.0)), (_L,))

    cx0 = _lane0_splat(0)
    cy0 = _lane0_splat(1)
    cz0 = _lane0_splat(2)

    fbase = jnp.float32(base)

    def _step(s, carry):
        cx, cy, cz, acc = carry

        neg1 = jnp.full((_L,), -1.0, jnp.float32)
        big = jnp.full((_L,), 1e9, jnp.float32)

        def _inner_body(g, accs):
            out = []
            fg = 64.0 * g.astype(jnp.float32)
            for u in range(4):
                bval, bidx = accs[u]
                off = base + g * (4 * _L) + u * _L
                x = xyz_v[0, pl.ds(off, _L)]
                y = xyz_v[1, pl.ds(off, _L)]
                z = xyz_v[2, pl.ds(off, _L)]
                dx = x - cx
                dy = y - cy
                dz = z - cz
                d = dx * dx + dy * dy
                d = d + dz * dz
                doff = g * (4 * _L) + u * _L
                nd = jnp.minimum(dists_v[pl.ds(doff, _L)], d)
                dists_v[pl.ds(doff, _L)] = nd
                idxv = f_iota + (fbase + fg + 16.0 * u)
                better = nd > bval
                bval = jnp.where(better, nd, bval)
                bidx = jnp.where(better, idxv, bidx)
                out.append((bval, bidx))
            return tuple(out)

        _inner = plsc.parallel_loop(
            0, _NVEC // 4, unroll=4,
            carry=((neg1, big), (neg1, big), (neg1, big), (neg1, big)),
        )(_inner_body)

        (bval, bidx) = _inner[0]
        for u in range(1, 4):
            bv, bi = _inner[u]
            bt = (bv > bval) | ((bv == bval) & (bi < bidx))
            bval = jnp.where(bt, bv, bval)
            bidx = jnp.where(bt, bi, bidx)

        rec_v[pl.ds(tz, _L)] = bval
        rec_v[pl.ds(tz + 16, _L)] = bidx
        p = lax.rem(s, 2)
        pltpu.sync_copy(rec_v.at[pl.ds(0, 32)],
                        shared.at[p, lb, pl.ds(t * 32, 32)])
        plsc.subcore_barrier()
        pltpu.sync_copy(shared.at[p, lb], comb_v)

        wv = comb_v[pl.ds(tz, _L)]
        wi = comb_v[pl.ds(tz + 16, _L)]
        for j in range(1, _TPB):
            bv = comb_v[pl.ds(tz + j * 32, _L)]
            bi = comb_v[pl.ds(tz + j * 32 + 16, _L)]
            better = (bv > wv) | ((bv == wv) & (bi < wi))
            wv = jnp.where(better, bv, wv)
            wi = jnp.where(better, bi, wi)

        m = jnp.max(wv)
        lidx_f = jnp.min(jnp.where(wv == m, wi, 1e9))
        offv = jnp.broadcast_to(lidx_f.astype(jnp.int32), (_L,))
        cxn = plsc.load_gather(xyz_v, [zeros_i, offv])
        cyn = plsc.load_gather(xyz_v, [ones_i, offv])
        czn = plsc.load_gather(xyz_v, [twos_i, offv])

        p16 = lax.rem(s, 16)
        acc = jnp.where(lane_iota == p16,
                        jnp.broadcast_to(lidx_f.astype(jnp.int32), (_L,)),
                        acc)

        @pl.when((t == 0) & (p16 == 15))
        def _():
            out_v[pl.ds(s - 15, _L)] = acc

        return (cxn, cyn, czn, acc)

    lax.fori_loop(1, _NPT, _step, (cx0, cy0, cz0, acc0))

    @pl.when(t == 0)
    def _():
        pltpu.sync_copy(out_v, out_hbm.at[b])


@functools.partial(
    pl.kernel,
    out_type=jax.ShapeDtypeStruct((_B, _NPT), jnp.int32),
    mesh=plsc.VectorSubcoreMesh(core_axis_name="c", subcore_axis_name="s"),
    compiler_params=pltpu.CompilerParams(needs_layout_passes=False),
    scratch_types=[
        pltpu.VMEM((3, _N), jnp.float32),
        pltpu.VMEM((_CHUNK,), jnp.float32),
        pltpu.VMEM((128,), jnp.float32),
        pltpu.VMEM((128,), jnp.float32),
        pltpu.VMEM((_NPT,), jnp.int32),
        pltpu.VMEM_SHARED((2, 4, 128), jnp.float32),
    ],
)
def _fps(xyz_hbm, out_hbm, xyz_v, dists_v, rec_v, comb_v, out_v, shared):
    _fps_body(xyz_hbm, out_hbm, xyz_v, dists_v, rec_v, comb_v, out_v, shared)


def kernel(xyz):
    return _fps(xyz.astype(jnp.float32))


# speedup vs baseline: 1.0092x; 1.0092x over previous
import functools

import jax
import jax.numpy as jnp
from jax import lax
from jax.experimental import pallas as pl
from jax.experimental.pallas import tpu as pltpu
from jax.experimental.pallas import tpu_sc as plsc

_B = 8
_N = 16384
_NPT = 1024
_TPB = 4
_CHUNK = _N // _TPB
_L = 16
_NVEC = _CHUNK // _L


def _fps_body(xyz_hbm, out_hbm, xyz_v, dists_v, rec_v, comb_v, out_v, shared):
    cid = lax.axis_index("c")
    sid = lax.axis_index("s")
    lb = sid // _TPB
    t = sid % _TPB
    b = cid * 4 + lb
    base = t * _CHUNK

    lane_iota = lax.iota(jnp.int32, _L)
    lane0 = lane_iota == 0
    f_iota = lane_iota.astype(jnp.float32)
    zeros_i = jnp.zeros((_L,), jnp.int32)
    ones_i = jnp.full((_L,), 1, jnp.int32)
    twos_i = jnp.full((_L,), 2, jnp.int32)

    tz = lax.shift_right_logical(sid, 5)

    pltpu.sync_copy(xyz_hbm.at[b], xyz_v)

    def _init(i, c):
        dists_v[pl.ds(i * _L, _L)] = jnp.full((_L,), 1e10, jnp.float32)
        return c
    lax.fori_loop(0, _NVEC, _init, 0)

    acc0 = jnp.zeros((_L,), jnp.int32)

    def _lane0_splat(row):
        v = xyz_v[row, pl.ds(tz, _L)]
        return jnp.broadcast_to(jnp.sum(jnp.where(lane0, v, 0.0)), (_L,))

    cx0 = _lane0_splat(0)
    cy0 = _lane0_splat(1)
    cz0 = _lane0_splat(2)

    fbase = jnp.float32(base)

    def _step(s, carry):
        cx, cy, cz, acc = carry

        neg1 = jnp.full((_L,), -1.0, jnp.float32)
        big = jnp.full((_L,), 1e9, jnp.float32)

        def _inner_body(g, accs):
            out = []
            fg = 128.0 * g.astype(jnp.float32)
            for u in range(8):
                bval, bidx = accs[u]
                off = base + g * (8 * _L) + u * _L
                x = xyz_v[0, pl.ds(off, _L)]
                y = xyz_v[1, pl.ds(off, _L)]
                z = xyz_v[2, pl.ds(off, _L)]
                dx = x - cx
                dy = y - cy
                dz = z - cz
                d = dx * dx + dy * dy
                d = d + dz * dz
                doff = g * (8 * _L) + u * _L
                nd = jnp.minimum(dists_v[pl.ds(doff, _L)], d)
                dists_v[pl.ds(doff, _L)] = nd
                idxv = f_iota + (fbase + fg + 16.0 * u)
                better = nd > bval
                bval = jnp.where(better, nd, bval)
                bidx = jnp.where(better, idxv, bidx)
                out.append((bval, bidx))
            return tuple(out)

        _inner = plsc.parallel_loop(
            0, _NVEC // 8, unroll=1,
            carry=tuple((neg1, big) for _ in range(8)),
        )(_inner_body)

        (bval, bidx) = _inner[0]
        for u in range(1, 8):
            bv, bi = _inner[u]
            bt = (bv > bval) | ((bv == bval) & (bi < bidx))
            bval = jnp.where(bt, bv, bval)
            bidx = jnp.where(bt, bi, bidx)

        rec_v[pl.ds(tz, _L)] = bval
        rec_v[pl.ds(tz + 16, _L)] = bidx
        p = lax.rem(s, 2)
        pltpu.sync_copy(rec_v.at[pl.ds(0, 32)],
                        shared.at[p, lb, pl.ds(t * 32, 32)])
        plsc.subcore_barrier()
        pltpu.sync_copy(shared.at[p, lb], comb_v)

        wv = comb_v[pl.ds(tz, _L)]
        wi = comb_v[pl.ds(tz + 16, _L)]
        for j in range(1, _TPB):
            bv = comb_v[pl.ds(tz + j * 32, _L)]
            bi = comb_v[pl.ds(tz + j * 32 + 16, _L)]
            better = (bv > wv) | ((bv == wv) & (bi < wi))
            wv = jnp.where(better, bv, wv)
            wi = jnp.where(better, bi, wi)

        m = jnp.max(wv)
        lidx_f = jnp.min(jnp.where(wv == m, wi, 1e9))
        offv = jnp.broadcast_to(lidx_f.astype(jnp.int32), (_L,))
        cxn = plsc.load_gather(xyz_v, [zeros_i, offv])
        cyn = plsc.load_gather(xyz_v, [ones_i, offv])
        czn = plsc.load_gather(xyz_v, [twos_i, offv])

        p16 = lax.rem(s, 16)
        acc = jnp.where(lane_iota == p16,
                        jnp.broadcast_to(lidx_f.astype(jnp.int32), (_L,)),
                        acc)

        @pl.when((t == 0) & (p16 == 15))
        def _():
            out_v[pl.ds(s - 15, _L)] = acc

        return (cxn, cyn, czn, acc)

    lax.fori_loop(1, _NPT, _step, (cx0, cy0, cz0, acc0))

    @pl.when(t == 0)
    def _():
        pltpu.sync_copy(out_v, out_hbm.at[b])


@functools.partial(
    pl.kernel,
    out_type=jax.ShapeDtypeStruct((_B, _NPT), jnp.int32),
    mesh=plsc.VectorSubcoreMesh(core_axis_name="c", subcore_axis_name="s"),
    compiler_params=pltpu.CompilerParams(needs_layout_passes=False),
    scratch_types=[
        pltpu.VMEM((3, _N), jnp.float32),
        pltpu.VMEM((_CHUNK,), jnp.float32),
        pltpu.VMEM((128,), jnp.float32),
        pltpu.VMEM((128,), jnp.float32),
        pltpu.VMEM((_NPT,), jnp.int32),
        pltpu.VMEM_SHARED((2, 4, 128), jnp.float32),
    ],
)
def _fps(xyz_hbm, out_hbm, xyz_v, dists_v, rec_v, comb_v, out_v, shared):
    _fps_body(xyz_hbm, out_hbm, xyz_v, dists_v, rec_v, comb_v, out_v, shared)


def kernel(xyz):
    return _fps(xyz.astype(jnp.float32))


# speedup vs baseline: 1.0590x; 1.0493x over previous
import functools

import jax
import jax.numpy as jnp
from jax import lax
from jax.experimental import pallas as pl
from jax.experimental.pallas import tpu as pltpu
from jax.experimental.pallas import tpu_sc as plsc

_B = 8
_N = 16384
_NPT = 1024
_TPB = 4
_CHUNK = _N // _TPB
_L = 16
_NVEC = _CHUNK // _L


def _fps_body(xyz_hbm, out_hbm, xyz_v, dists_v, rec_v, comb_v, out_v, shared):
    cid = lax.axis_index("c")
    sid = lax.axis_index("s")
    lb = sid // _TPB
    t = sid % _TPB
    b = cid * 4 + lb
    base = t * _CHUNK

    lane_iota = lax.iota(jnp.int32, _L)
    lane0 = lane_iota == 0
    f_iota = lane_iota.astype(jnp.float32)
    zeros_i = jnp.zeros((_L,), jnp.int32)
    ones_i = jnp.full((_L,), 1, jnp.int32)
    twos_i = jnp.full((_L,), 2, jnp.int32)

    tz = lax.shift_right_logical(sid, 5)

    pltpu.sync_copy(xyz_hbm.at[b], xyz_v)

    def _init(i, c):
        dists_v[pl.ds(i * _L, _L)] = jnp.full((_L,), 1e10, jnp.float32)
        return c
    lax.fori_loop(0, _NVEC, _init, 0)

    acc0 = jnp.zeros((_L,), jnp.int32)

    def _lane0_splat(row):
        v = xyz_v[row, pl.ds(tz, _L)]
        return jnp.broadcast_to(jnp.sum(jnp.where(lane0, v, 0.0)), (_L,))

    cx0 = _lane0_splat(0)
    cy0 = _lane0_splat(1)
    cz0 = _lane0_splat(2)

    fbase = jnp.float32(base)

    def _step(s, carry):
        cx, cy, cz, acc = carry

        neg1 = jnp.full((_L,), -1.0, jnp.float32)
        big = jnp.full((_L,), 1e9, jnp.float32)

        def _inner_body(g, accs):
            out = []
            fg = 64.0 * g.astype(jnp.float32)
            for u in range(4):
                bval, bidx = accs[u]
                off = base + g * (4 * _L) + u * _L
                x = xyz_v[0, pl.ds(off, _L)]
                y = xyz_v[1, pl.ds(off, _L)]
                z = xyz_v[2, pl.ds(off, _L)]
                dx = x - cx
                dy = y - cy
                dz = z - cz
                d = dx * dx + dy * dy
                d = d + dz * dz
                doff = g * (4 * _L) + u * _L
                nd = jnp.minimum(dists_v[pl.ds(doff, _L)], d)
                dists_v[pl.ds(doff, _L)] = nd
                idxv = f_iota + (fbase + fg + 16.0 * u)
                better = nd > bval
                bval = jnp.where(better, nd, bval)
                bidx = jnp.where(better, idxv, bidx)
                out.append((bval, bidx))
            return tuple(out)

        _inner = plsc.parallel_loop(
            0, _NVEC // 4, unroll=2,
            carry=tuple((neg1, big) for _ in range(4)),
        )(_inner_body)

        (bval, bidx) = _inner[0]
        for u in range(1, 4):
            bv, bi = _inner[u]
            bt = (bv > bval) | ((bv == bval) & (bi < bidx))
            bval = jnp.where(bt, bv, bval)
            bidx = jnp.where(bt, bi, bidx)

        rec_v[pl.ds(tz, _L)] = bval
        rec_v[pl.ds(tz + 16, _L)] = bidx
        p = lax.rem(s, 2)
        pltpu.sync_copy(rec_v.at[pl.ds(0, 32)],
                        shared.at[p, lb, pl.ds(t * 32, 32)])
        plsc.subcore_barrier()
        pltpu.sync_copy(shared.at[p, lb], comb_v)

        wv = comb_v[pl.ds(tz, _L)]
        wi = comb_v[pl.ds(tz + 16, _L)]
        for j in range(1, _TPB):
            bv = comb_v[pl.ds(tz + j * 32, _L)]
            bi = comb_v[pl.ds(tz + j * 32 + 16, _L)]
            better = (bv > wv) | ((bv == wv) & (bi < wi))
            wv = jnp.where(better, bv, wv)
            wi = jnp.where(better, bi, wi)

        m = jnp.max(wv)
        lidx_f = jnp.min(jnp.where(wv == m, wi, 1e9))
        offv = jnp.broadcast_to(lidx_f.astype(jnp.int32), (_L,))
        cxn = plsc.load_gather(xyz_v, [zeros_i, offv])
        cyn = plsc.load_gather(xyz_v, [ones_i, offv])
        czn = plsc.load_gather(xyz_v, [twos_i, offv])

        p16 = lax.rem(s, 16)
        acc = jnp.where(lane_iota == p16,
                        jnp.broadcast_to(lidx_f.astype(jnp.int32), (_L,)),
                        acc)

        @pl.when((t == 0) & (p16 == 15))
        def _():
            out_v[pl.ds(s - 15, _L)] = acc

        return (cxn, cyn, czn, acc)

    lax.fori_loop(1, _NPT, _step, (cx0, cy0, cz0, acc0))

    @pl.when(t == 0)
    def _():
        pltpu.sync_copy(out_v, out_hbm.at[b])


@functools.partial(
    pl.kernel,
    out_type=jax.ShapeDtypeStruct((_B, _NPT), jnp.int32),
    mesh=plsc.VectorSubcoreMesh(core_axis_name="c", subcore_axis_name="s"),
    compiler_params=pltpu.CompilerParams(needs_layout_passes=False),
    scratch_types=[
        pltpu.VMEM((3, _N), jnp.float32),
        pltpu.VMEM((_CHUNK,), jnp.float32),
        pltpu.VMEM((128,), jnp.float32),
        pltpu.VMEM((128,), jnp.float32),
        pltpu.VMEM((_NPT,), jnp.int32),
        pltpu.VMEM_SHARED((2, 4, 128), jnp.float32),
    ],
)
def _fps(xyz_hbm, out_hbm, xyz_v, dists_v, rec_v, comb_v, out_v, shared):
    _fps_body(xyz_hbm, out_hbm, xyz_v, dists_v, rec_v, comb_v, out_v, shared)


def kernel(xyz):
    return _fps(xyz.astype(jnp.float32))


# speedup vs baseline: 1.1236x; 1.0611x over previous
import functools

import jax
import jax.numpy as jnp
from jax import lax
from jax.experimental import pallas as pl
from jax.experimental.pallas import tpu as pltpu
from jax.experimental.pallas import tpu_sc as plsc

_B = 8
_N = 16384
_NPT = 1024
_TPB = 4
_CHUNK = _N // _TPB
_L = 16
_NVEC = _CHUNK // _L


def _fps_body(xyz_hbm, out_hbm, xyz_v, dists_v, rec_v, comb_v, out_v, shared):
    cid = lax.axis_index("c")
    sid = lax.axis_index("s")
    lb = sid // _TPB
    t = sid % _TPB
    b = cid * 4 + lb
    base = t * _CHUNK

    lane_iota = lax.iota(jnp.int32, _L)
    lane0 = lane_iota == 0
    f_iota = lane_iota.astype(jnp.float32)
    zeros_i = jnp.zeros((_L,), jnp.int32)
    ones_i = jnp.full((_L,), 1, jnp.int32)
    twos_i = jnp.full((_L,), 2, jnp.int32)

    tz = lax.shift_right_logical(sid, 5)

    pltpu.sync_copy(xyz_hbm.at[b], xyz_v)

    def _init(i, c):
        dists_v[pl.ds(i * _L, _L)] = jnp.full((_L,), 1e10, jnp.float32)
        return c
    lax.fori_loop(0, _NVEC, _init, 0)

    acc0 = jnp.zeros((_L,), jnp.int32)

    def _lane0_splat(row):
        v = xyz_v[row, pl.ds(tz, _L)]
        return jnp.broadcast_to(jnp.sum(jnp.where(lane0, v, 0.0)), (_L,))

    cx0 = _lane0_splat(0)
    cy0 = _lane0_splat(1)
    cz0 = _lane0_splat(2)

    fbase = jnp.float32(base)

    def _step(s, carry):
        cx, cy, cz, acc = carry

        neg1 = jnp.full((_L,), -1.0, jnp.float32)
        big = jnp.full((_L,), 1e9, jnp.float32)

        def _inner_body(g, accs):
            out = []
            fg = 64.0 * g.astype(jnp.float32)
            for u in range(4):
                bval, bidx = accs[u]
                off = base + g * (4 * _L) + u * _L
                x = xyz_v[0, pl.ds(off, _L)]
                y = xyz_v[1, pl.ds(off, _L)]
                z = xyz_v[2, pl.ds(off, _L)]
                dx = x - cx
                dy = y - cy
                dz = z - cz
                d = dx * dx + dy * dy
                d = d + dz * dz
                doff = g * (4 * _L) + u * _L
                nd = jnp.minimum(dists_v[pl.ds(doff, _L)], d)
                dists_v[pl.ds(doff, _L)] = nd
                sidx = fbase + fg + 16.0 * u
                better = nd > bval
                bval = jnp.where(better, nd, bval)
                bidx = jnp.where(better, jnp.broadcast_to(sidx, (_L,)), bidx)
                out.append((bval, bidx))
            return tuple(out)

        _inner = plsc.parallel_loop(
            0, _NVEC // 4, unroll=2,
            carry=tuple((neg1, big) for _ in range(4)),
        )(_inner_body)

        (bval, bidx) = _inner[0]
        for u in range(1, 4):
            bv, bi = _inner[u]
            bt = (bv > bval) | ((bv == bval) & (bi < bidx))
            bval = jnp.where(bt, bv, bval)
            bidx = jnp.where(bt, bi, bidx)
        bidx = bidx + f_iota

        rec_v[pl.ds(tz, _L)] = bval
        rec_v[pl.ds(tz + 16, _L)] = bidx
        p = lax.rem(s, 2)
        pltpu.sync_copy(rec_v.at[pl.ds(0, 32)],
                        shared.at[p, lb, pl.ds(t * 32, 32)])
        plsc.subcore_barrier()
        pltpu.sync_copy(shared.at[p, lb], comb_v)

        wv = comb_v[pl.ds(tz, _L)]
        wi = comb_v[pl.ds(tz + 16, _L)]
        for j in range(1, _TPB):
            bv = comb_v[pl.ds(tz + j * 32, _L)]
            bi = comb_v[pl.ds(tz + j * 32 + 16, _L)]
            better = (bv > wv) | ((bv == wv) & (bi < wi))
            wv = jnp.where(better, bv, wv)
            wi = jnp.where(better, bi, wi)

        m = jnp.max(wv)
        lidx_f = jnp.min(jnp.where(wv == m, wi, 1e9))
        offv = jnp.broadcast_to(lidx_f.astype(jnp.int32), (_L,))
        cxn = plsc.load_gather(xyz_v, [zeros_i, offv])
        cyn = plsc.load_gather(xyz_v, [ones_i, offv])
        czn = plsc.load_gather(xyz_v, [twos_i, offv])

        p16 = lax.rem(s, 16)
        acc = jnp.where(lane_iota == p16,
                        jnp.broadcast_to(lidx_f.astype(jnp.int32), (_L,)),
                        acc)

        @pl.when((t == 0) & (p16 == 15))
        def _():
            out_v[pl.ds(s - 15, _L)] = acc

        return (cxn, cyn, czn, acc)

    lax.fori_loop(1, _NPT, _step, (cx0, cy0, cz0, acc0))

    @pl.when(t == 0)
    def _():
        pltpu.sync_copy(out_v, out_hbm.at[b])


@functools.partial(
    pl.kernel,
    out_type=jax.ShapeDtypeStruct((_B, _NPT), jnp.int32),
    mesh=plsc.VectorSubcoreMesh(core_axis_name="c", subcore_axis_name="s"),
    compiler_params=pltpu.CompilerParams(needs_layout_passes=False),
    scratch_types=[
        pltpu.VMEM((3, _N), jnp.float32),
        pltpu.VMEM((_CHUNK,), jnp.float32),
        pltpu.VMEM((128,), jnp.float32),
        pltpu.VMEM((128,), jnp.float32),
        pltpu.VMEM((_NPT,), jnp.int32),
        pltpu.VMEM_SHARED((2, 4, 128), jnp.float32),
    ],
)
def _fps(xyz_hbm, out_hbm, xyz_v, dists_v, rec_v, comb_v, out_v, shared):
    _fps_body(xyz_hbm, out_hbm, xyz_v, dists_v, rec_v, comb_v, out_v, shared)


def kernel(xyz):
    return _fps(xyz.astype(jnp.float32))
